# Initial kernel scaffold; baseline (speedup 1.0000x reference)
import jax
import jax.numpy as jnp
from jax.experimental import pallas as pl


def kernel(positions, node_feats, senders, receivers, cutoff, W_up, W1, W2, W_down):
    raise NotImplementedError("write your pallas kernel here")


# speedup vs baseline: 3.2484x; 3.2484x over previous
import math

import jax
import jax.numpy as jnp
from jax import lax
from jax.experimental import pallas as pl
from jax.experimental.pallas import tpu as pltpu
from jax.experimental.pallas import tpu_sc as plsc

_N = 10000
_E = 160000
_D = 128
_NRB = 8
_HID = 64
_MSG = 4 * _D
_AVG = 16.0

_NC = 2
_NS = 16
_NW = _NC * _NS
_GCH = 128
_NCHUNK = _E // _GCH
_JMAX = (_NCHUNK + _NW - 1) // _NW
_NPAD = 10240
_NPT = _NPAD // _NS

_PCOLS = 16


def _mesh():
    return plsc.VectorSubcoreMesh(core_axis_name="c", subcore_axis_name="s",
                                  num_cores=_NC, num_subcores=_NS)


def _sc_gather_body(snd, rcv, nf, posx, posy, posz,
                    msg_out, vx_out, vy_out, vz_out,
                    sidx, ridx, rows, dxb, dyb, dzb, px, py, pz, sem):
    w = lax.axis_index("s") * _NC + lax.axis_index("c")
    pltpu.sync_copy(posx, px)
    pltpu.sync_copy(posy, py)
    pltpu.sync_copy(posz, pz)

    def chunk(j, carry):
        cid = j * _NW + w

        @pl.when(cid < _NCHUNK)
        def _():
            off = cid * _GCH
            pltpu.sync_copy(snd.at[pl.ds(off, _GCH)], sidx)
            pltpu.sync_copy(rcv.at[pl.ds(off, _GCH)], ridx)
            cp = pltpu.async_copy(nf.at[sidx], rows, sem)
            for g in range(_GCH // 16):
                sl = pl.ds(g * 16, 16)
                iv_s = sidx[sl]
                iv_r = ridx[sl]
                dxb[sl] = (plsc.load_gather(px, [iv_r])
                           - plsc.load_gather(px, [iv_s]))
                dyb[sl] = (plsc.load_gather(py, [iv_r])
                           - plsc.load_gather(py, [iv_s]))
                dzb[sl] = (plsc.load_gather(pz, [iv_r])
                           - plsc.load_gather(pz, [iv_s]))
            cp.wait()
            pltpu.sync_copy(rows, msg_out.at[pl.ds(off, _GCH), :])
            pltpu.sync_copy(dxb, vx_out.at[pl.ds(off, _GCH)])
            pltpu.sync_copy(dyb, vy_out.at[pl.ds(off, _GCH)])
            pltpu.sync_copy(dzb, vz_out.at[pl.ds(off, _GCH)])

        return carry

    lax.fori_loop(0, _JMAX, chunk, 0)


def _make_sc_gather():
    return pl.kernel(
        _sc_gather_body,
        out_type=(
            jax.ShapeDtypeStruct((_E, _D), jnp.float32),
            jax.ShapeDtypeStruct((_E,), jnp.float32),
            jax.ShapeDtypeStruct((_E,), jnp.float32),
            jax.ShapeDtypeStruct((_E,), jnp.float32),
        ),
        mesh=_mesh(),
        scratch_types=[
            pltpu.VMEM((_GCH,), jnp.int32),
            pltpu.VMEM((_GCH,), jnp.int32),
            pltpu.VMEM((_GCH, _D), jnp.float32),
            pltpu.VMEM((_GCH,), jnp.float32),
            pltpu.VMEM((_GCH,), jnp.float32),
            pltpu.VMEM((_GCH,), jnp.float32),
            pltpu.VMEM((_N,), jnp.float32),
            pltpu.VMEM((_N,), jnp.float32),
            pltpu.VMEM((_N,), jnp.float32),
            pltpu.SemaphoreType.DMA,
        ],
        compiler_params=pltpu.CompilerParams(needs_layout_passes=False),
    )


def _sc_scatter_body(contrib, rcv, zeros, out, idxb, rowsb, accum):
    c = lax.axis_index("c")
    s = lax.axis_index("s")
    w = s * _NC + c
    pltpu.sync_copy(zeros.at[pl.ds(s * _NPT, _NPT), :],
                    accum.at[pl.ds(s * _NPT, _NPT), :])
    plsc.subcore_barrier()

    def chunk(j, carry):
        cid = j * _NW + w

        @pl.when(cid < _NCHUNK)
        def _():
            off = cid * _GCH
            pltpu.sync_copy(rcv.at[pl.ds(off, _GCH)], idxb)
            pltpu.sync_copy(contrib.at[pl.ds(off, _GCH), :], rowsb)
            pltpu.sync_copy(rowsb, accum.at[idxb], add=True)

        return carry

    lax.fori_loop(0, _JMAX, chunk, 0)
    plsc.subcore_barrier()
    pltpu.sync_copy(accum.at[pl.ds(s * _NPT, _NPT), :],
                    out.at[c, pl.ds(s * _NPT, _NPT), :])


def _make_sc_scatter():
    return pl.kernel(
        _sc_scatter_body,
        out_type=jax.ShapeDtypeStruct((_NC, _NPAD, _D), jnp.float32),
        mesh=_mesh(),
        scratch_types=[
            pltpu.VMEM((_GCH,), jnp.int32),
            pltpu.VMEM((_GCH, _D), jnp.float32),
            pltpu.VMEM_SHARED((_NPAD, _D), jnp.float32),
        ],
    )


def _tc_up_body(x_ref, w_ref, o_ref):
    o_ref[...] = jnp.dot(x_ref[...], w_ref[...],
                         preferred_element_type=jnp.float32) * (1.0 / math.sqrt(_D))


def _tc_edge_body(cut_ref, msg_ref, vx_ref, vy_ref, vz_ref,
                  w1_ref, w2_ref, wd_ref, o_ref):
    cutoff = cut_ref[0]
    vx = vx_ref[...]
    vy = vy_ref[...]
    vz = vz_ref[...]
    r = jnp.sqrt(vx * vx + vy * vy + vz * vz)

    ks = lax.broadcasted_iota(jnp.int32, (1, _NRB), 1).astype(jnp.float32) + 1.0
    safe = jnp.where(r > 1e-9, r, 1e-9)
    sq2c = jnp.sqrt(2.0 / cutoff)
    b = sq2c * jnp.sin(ks * (jnp.pi / cutoff) * r) / safe
    lim = sq2c * ks * (jnp.pi / cutoff)
    radial = jnp.where(r > 1e-9, b, lim)

    cste = 1.2 / math.exp(-1.0 / 2.0)
    y = 2.0 * (1.0 - r / cutoff)
    ysafe = jnp.where(y > 0.0, y, 1.0)
    env = cste * jnp.where(y > 0.0, jnp.exp(-1.0 / ysafe), 0.0)
    radial = radial * env

    h = jax.nn.gelu(jnp.dot(radial, w1_ref[...],
                            preferred_element_type=jnp.float32)
                    * (1.0 / math.sqrt(_NRB)))
    mix = jnp.dot(h, w2_ref[...],
                  preferred_element_type=jnp.float32) * (1.0 / math.sqrt(_HID))

    rs = math.sqrt(3.0) / jnp.where(r > 1e-9, r, 1.0)
    m = msg_ref[...]
    a = jnp.concatenate([
        m * mix[:, 0 * _D:1 * _D],
        m * mix[:, 1 * _D:2 * _D] * (vx * rs),
        m * mix[:, 2 * _D:3 * _D] * (vy * rs),
        m * mix[:, 3 * _D:4 * _D] * (vz * rs),
    ], axis=1)
    o_ref[...] = jnp.dot(a, wd_ref[...],
                         preferred_element_type=jnp.float32) * (
        1.0 / (math.sqrt(_MSG) * math.sqrt(_AVG)))


def _tc_add_body(p_ref, o_ref):
    o_ref[...] = p_ref[0] + p_ref[1]


_BN = 2000
_BE = 1000


def _tc_up(x, w):
    return pl.pallas_call(
        _tc_up_body,
        grid=(_N // _BN,),
        in_specs=[
            pl.BlockSpec((_BN, _D), lambda i: (i, 0)),
            pl.BlockSpec((_D, _D), lambda i: (0, 0)),
        ],
        out_specs=pl.BlockSpec((_BN, _D), lambda i: (i, 0)),
        out_shape=jax.ShapeDtypeStruct((_N, _D), jnp.float32),
        compiler_params=pltpu.CompilerParams(
            dimension_semantics=("parallel",)),
    )(x, w)


def _tc_edge(cut, msg, vx, vy, vz, w1, w2p, wdp):
    return pl.pallas_call(
        _tc_edge_body,
        grid=(_E // _BE,),
        in_specs=[
            pl.BlockSpec(memory_space=pltpu.SMEM),
            pl.BlockSpec((_BE, _D), lambda i: (i, 0)),
            pl.BlockSpec((_BE, 1), lambda i: (i, 0)),
            pl.BlockSpec((_BE, 1), lambda i: (i, 0)),
            pl.BlockSpec((_BE, 1), lambda i: (i, 0)),
            pl.BlockSpec((_NRB, _HID), lambda i: (0, 0)),
            pl.BlockSpec((_HID, _MSG), lambda i: (0, 0)),
            pl.BlockSpec((_MSG, _D), lambda i: (0, 0)),
        ],
        out_specs=pl.BlockSpec((_BE, _D), lambda i: (i, 0)),
        out_shape=jax.ShapeDtypeStruct((_E, _D), jnp.float32),
        compiler_params=pltpu.CompilerParams(
            dimension_semantics=("parallel",)),
    )(cut, msg, vx, vy, vz, w1, w2p, wdp)


def _tc_add(p):
    return pl.pallas_call(
        _tc_add_body,
        grid=(_N // _BN,),
        in_specs=[pl.BlockSpec((_NC, _BN, _D), lambda i: (0, i, 0))],
        out_specs=pl.BlockSpec((_BN, _D), lambda i: (i, 0)),
        out_shape=jax.ShapeDtypeStruct((_N, _D), jnp.float32),
        compiler_params=pltpu.CompilerParams(
            dimension_semantics=("parallel",)),
    )(p)


def kernel(positions, node_feats, senders, receivers, cutoff,
           W_up, W1, W2, W_down):
    f32 = jnp.float32
    posx = positions[:, 0].astype(f32)
    posy = positions[:, 1].astype(f32)
    posz = positions[:, 2].astype(f32)
    s32 = senders.astype(jnp.int32)
    r32 = receivers.astype(jnp.int32)

    t2 = W2[:, _D:].reshape(_HID, _D, 3)
    w2p = jnp.concatenate([W2[:, :_D], t2[:, :, 0], t2[:, :, 1], t2[:, :, 2]],
                          axis=1).astype(f32)
    td = W_down[_D:, :].reshape(_D, 3, _D)
    wdp = jnp.concatenate([W_down[:_D, :], td[:, 0, :], td[:, 1, :],
                           td[:, 2, :]], axis=0).astype(f32)

    nf = _tc_up(node_feats.astype(f32), W_up.astype(f32))
    msg, vx, vy, vz = _make_sc_gather()(s32, r32, nf, posx, posy, posz)
    cut = jnp.asarray(cutoff, f32).reshape(1)
    contrib = _tc_edge(cut, msg, vx.reshape(_E, 1), vy.reshape(_E, 1),
                       vz.reshape(_E, 1), W1.astype(f32), w2p, wdp)
    zeros = jnp.zeros((_NPAD, _D), f32)
    partials = _make_sc_scatter()(contrib, r32, zeros)
    return _tc_add(partials)


# speedup vs baseline: 7.2459x; 2.2306x over previous
import math

import jax
import jax.numpy as jnp
from jax import lax
from jax.experimental import pallas as pl
from jax.experimental.pallas import tpu as pltpu
from jax.experimental.pallas import tpu_sc as plsc

_N = 10000
_E = 160000
_D = 128
_NRB = 8
_HID = 64
_MSG = 4 * _D
_AVG = 16.0

_NC = 2
_NS = 16
_NW = _NC * _NS
_GCH = 128
_NCHUNK = _E // _GCH
_JMAX = (_NCHUNK + _NW - 1) // _NW
_NPAD = 10240
_NPT = _NPAD // _NS

_PCOLS = 16


def _mesh():
    return plsc.VectorSubcoreMesh(core_axis_name="c", subcore_axis_name="s",
                                  num_cores=_NC, num_subcores=_NS)


def _sc_gather_body(snd, rcv, nf, posx, posy, posz,
                    msg_out, vx_out, vy_out, vz_out,
                    sidx, ridx, rows, dxb, dyb, dzb, px, py, pz, sem):
    w = lax.axis_index("s") * _NC + lax.axis_index("c")
    pltpu.sync_copy(posx, px)
    pltpu.sync_copy(posy, py)
    pltpu.sync_copy(posz, pz)

    def chunk(j, carry):
        cid = j * _NW + w

        @pl.when(cid < _NCHUNK)
        def _():
            off = cid * _GCH
            pltpu.sync_copy(snd.at[pl.ds(off, _GCH)], sidx)
            pltpu.sync_copy(rcv.at[pl.ds(off, _GCH)], ridx)
            cp = pltpu.async_copy(nf.at[sidx], rows, sem)
            for g in range(_GCH // 16):
                sl = pl.ds(g * 16, 16)
                iv_s = sidx[sl]
                iv_r = ridx[sl]
                dxb[sl] = (plsc.load_gather(px, [iv_r])
                           - plsc.load_gather(px, [iv_s]))
                dyb[sl] = (plsc.load_gather(py, [iv_r])
                           - plsc.load_gather(py, [iv_s]))
                dzb[sl] = (plsc.load_gather(pz, [iv_r])
                           - plsc.load_gather(pz, [iv_s]))
            cp.wait()
            pltpu.sync_copy(rows, msg_out.at[pl.ds(off, _GCH), :])
            pltpu.sync_copy(dxb, vx_out.at[cid, 0])
            pltpu.sync_copy(dyb, vy_out.at[cid, 0])
            pltpu.sync_copy(dzb, vz_out.at[cid, 0])

        return carry

    lax.fori_loop(0, _JMAX, chunk, 0)


def _make_sc_gather():
    return pl.kernel(
        _sc_gather_body,
        out_type=(
            jax.ShapeDtypeStruct((_E, _D), jnp.float32),
            jax.ShapeDtypeStruct((_NCHUNK, 1, _GCH), jnp.float32),
            jax.ShapeDtypeStruct((_NCHUNK, 1, _GCH), jnp.float32),
            jax.ShapeDtypeStruct((_NCHUNK, 1, _GCH), jnp.float32),
        ),
        mesh=_mesh(),
        scratch_types=[
            pltpu.VMEM((_GCH,), jnp.int32),
            pltpu.VMEM((_GCH,), jnp.int32),
            pltpu.VMEM((_GCH, _D), jnp.float32),
            pltpu.VMEM((_GCH,), jnp.float32),
            pltpu.VMEM((_GCH,), jnp.float32),
            pltpu.VMEM((_GCH,), jnp.float32),
            pltpu.VMEM((_N,), jnp.float32),
            pltpu.VMEM((_N,), jnp.float32),
            pltpu.VMEM((_N,), jnp.float32),
            pltpu.SemaphoreType.DMA,
        ],
        compiler_params=pltpu.CompilerParams(needs_layout_passes=False),
    )


def _sc_scatter_body(contrib, rcv, zeros, out, idxb, rowsb, accum):
    c = lax.axis_index("c")
    s = lax.axis_index("s")
    w = s * _NC + c
    pltpu.sync_copy(zeros.at[pl.ds(s * _NPT, _NPT), :],
                    accum.at[pl.ds(s * _NPT, _NPT), :])
    plsc.subcore_barrier()

    def chunk(j, carry):
        cid = j * _NW + w

        @pl.when(cid < _NCHUNK)
        def _():
            off = cid * _GCH
            pltpu.sync_copy(rcv.at[pl.ds(off, _GCH)], idxb)
            pltpu.sync_copy(contrib.at[pl.ds(off, _GCH), :], rowsb)
            pltpu.sync_copy(rowsb, accum.at[idxb], add=True)

        return carry

    lax.fori_loop(0, _JMAX, chunk, 0)
    plsc.subcore_barrier()
    pltpu.sync_copy(accum.at[pl.ds(s * _NPT, _NPT), :],
                    out.at[c, pl.ds(s * _NPT, _NPT), :])


def _make_sc_scatter():
    return pl.kernel(
        _sc_scatter_body,
        out_type=jax.ShapeDtypeStruct((_NC, _NPAD, _D), jnp.float32),
        mesh=_mesh(),
        scratch_types=[
            pltpu.VMEM((_GCH,), jnp.int32),
            pltpu.VMEM((_GCH, _D), jnp.float32),
            pltpu.VMEM_SHARED((_NPAD, _D), jnp.float32),
        ],
    )


def _tc_up_body(x_ref, w_ref, o_ref):
    o_ref[...] = jnp.dot(x_ref[...], w_ref[...],
                         preferred_element_type=jnp.float32) * (1.0 / math.sqrt(_D))


def _tc_edge_body(cut_ref, msg_ref, vx_ref, vy_ref, vz_ref,
                  w1t_ref, w2t_ref, wdt_ref, o_ref):
    cutoff = cut_ref[0]
    nrow = _BE // _GCH
    vxl = vx_ref[:, 0, :]
    vyl = vy_ref[:, 0, :]
    vzl = vz_ref[:, 0, :]
    rl = jnp.sqrt(vxl * vxl + vyl * vyl + vzl * vzl)
    ul = math.sqrt(3.0) / jnp.where(rl > 1e-9, rl, 1.0)
    uxl = vxl * ul
    uyl = vyl * ul
    uzl = vzl * ul
    rrow = jnp.concatenate([rl[j:j + 1, :] for j in range(nrow)], axis=1)
    ux = jnp.concatenate([uxl[j:j + 1, :] for j in range(nrow)], axis=1)
    uy = jnp.concatenate([uyl[j:j + 1, :] for j in range(nrow)], axis=1)
    uz = jnp.concatenate([uzl[j:j + 1, :] for j in range(nrow)], axis=1)

    ks = lax.broadcasted_iota(jnp.int32, (_NRB, 1), 0).astype(jnp.float32) + 1.0
    safe = jnp.where(rrow > 1e-9, rrow, 1e-9)
    sq2c = jnp.sqrt(2.0 / cutoff)
    b = sq2c * jnp.sin(ks * (jnp.pi / cutoff) * rrow) / safe
    lim = (sq2c * (jnp.pi / cutoff)) * ks
    radial = jnp.where(rrow > 1e-9, b, jnp.broadcast_to(lim, b.shape))

    cste = 1.2 / math.exp(-1.0 / 2.0)
    y = 2.0 * (1.0 - rrow / cutoff)
    ysafe = jnp.where(y > 0.0, y, 1.0)
    env = cste * jnp.where(y > 0.0, jnp.exp(-1.0 / ysafe), 0.0)
    radial = radial * env

    h = jax.nn.gelu(jnp.dot(w1t_ref[...], radial,
                            preferred_element_type=jnp.float32)
                    * (1.0 / math.sqrt(_NRB)))
    mix = jnp.dot(w2t_ref[...], h,
                  preferred_element_type=jnp.float32) * (1.0 / math.sqrt(_HID))

    mt = msg_ref[...].T
    at = jnp.concatenate([
        mt * mix[0 * _D:1 * _D, :],
        mt * mix[1 * _D:2 * _D, :] * ux,
        mt * mix[2 * _D:3 * _D, :] * uy,
        mt * mix[3 * _D:4 * _D, :] * uz,
    ], axis=0)
    ct = jnp.dot(wdt_ref[...], at,
                 preferred_element_type=jnp.float32) * (
        1.0 / (math.sqrt(_MSG) * math.sqrt(_AVG)))
    o_ref[...] = ct.T


def _tc_add_body(p_ref, o_ref):
    o_ref[...] = p_ref[0] + p_ref[1]


_BN = 2000
_BE = 1280


def _tc_up(x, w):
    return pl.pallas_call(
        _tc_up_body,
        grid=(_N // _BN,),
        in_specs=[
            pl.BlockSpec((_BN, _D), lambda i: (i, 0)),
            pl.BlockSpec((_D, _D), lambda i: (0, 0)),
        ],
        out_specs=pl.BlockSpec((_BN, _D), lambda i: (i, 0)),
        out_shape=jax.ShapeDtypeStruct((_N, _D), jnp.float32),
        compiler_params=pltpu.CompilerParams(
            dimension_semantics=("parallel",)),
    )(x, w)


def _tc_edge(cut, msg, vx, vy, vz, w1, w2p, wdp):
    return pl.pallas_call(
        _tc_edge_body,
        grid=(_E // _BE,),
        in_specs=[
            pl.BlockSpec(memory_space=pltpu.SMEM),
            pl.BlockSpec((_BE, _D), lambda i: (i, 0)),
            pl.BlockSpec((_BE // _GCH, 1, _GCH), lambda i: (i, 0, 0)),
            pl.BlockSpec((_BE // _GCH, 1, _GCH), lambda i: (i, 0, 0)),
            pl.BlockSpec((_BE // _GCH, 1, _GCH), lambda i: (i, 0, 0)),
            pl.BlockSpec((_HID, _NRB), lambda i: (0, 0)),
            pl.BlockSpec((_MSG, _HID), lambda i: (0, 0)),
            pl.BlockSpec((_D, _MSG), lambda i: (0, 0)),
        ],
        out_specs=pl.BlockSpec((_BE, _D), lambda i: (i, 0)),
        out_shape=jax.ShapeDtypeStruct((_E, _D), jnp.float32),
        compiler_params=pltpu.CompilerParams(
            dimension_semantics=("parallel",)),
    )(cut, msg, vx, vy, vz, w1, w2p, wdp)


def _tc_add(p):
    return pl.pallas_call(
        _tc_add_body,
        grid=(_N // _BN,),
        in_specs=[pl.BlockSpec((_NC, _BN, _D), lambda i: (0, i, 0))],
        out_specs=pl.BlockSpec((_BN, _D), lambda i: (i, 0)),
        out_shape=jax.ShapeDtypeStruct((_N, _D), jnp.float32),
        compiler_params=pltpu.CompilerParams(
            dimension_semantics=("parallel",)),
    )(p)


def kernel(positions, node_feats, senders, receivers, cutoff,
           W_up, W1, W2, W_down):
    f32 = jnp.float32
    posx = positions[:, 0].astype(f32)
    posy = positions[:, 1].astype(f32)
    posz = positions[:, 2].astype(f32)
    s32 = senders.astype(jnp.int32)
    r32 = receivers.astype(jnp.int32)

    t2 = W2[:, _D:].reshape(_HID, _D, 3)
    w2p = jnp.concatenate([W2[:, :_D], t2[:, :, 0], t2[:, :, 1], t2[:, :, 2]],
                          axis=1).astype(f32)
    td = W_down[_D:, :].reshape(_D, 3, _D)
    wdp = jnp.concatenate([W_down[:_D, :], td[:, 0, :], td[:, 1, :],
                           td[:, 2, :]], axis=0).astype(f32)

    nf = _tc_up(node_feats.astype(f32), W_up.astype(f32))
    msg, vx, vy, vz = _make_sc_gather()(s32, r32, nf, posx, posy, posz)
    cut = jnp.asarray(cutoff, f32).reshape(1)
    contrib = _tc_edge(cut, msg, vx, vy, vz, W1.T.astype(f32), w2p.T, wdp.T)
    zeros = jnp.zeros((_NPAD, _D), f32)
    partials = _make_sc_scatter()(contrib, r32, zeros)
    return _tc_add(partials)


# speedup vs baseline: 8.0778x; 1.1148x over previous
import math

import jax
import jax.numpy as jnp
from jax import lax
from jax.experimental import pallas as pl
from jax.experimental.pallas import tpu as pltpu
from jax.experimental.pallas import tpu_sc as plsc

_N = 10000
_E = 160000
_D = 128
_NRB = 8
_HID = 64
_MSG = 4 * _D
_AVG = 16.0

_NC = 2
_NS = 16
_NW = _NC * _NS
_GCH = 128
_BCH = 5
_BED = _BCH * _GCH
_NBATCH = 252
_EP = _NBATCH * _BED
_NCHUNK = _EP // _GCH
_JMAX = (_NBATCH + _NW - 1) // _NW
_NPAD = 10240
_NPT = _NPAD // _NS


def _mesh():
    return plsc.VectorSubcoreMesh(core_axis_name="c", subcore_axis_name="s",
                                  num_cores=_NC, num_subcores=_NS)


def _sc_gather_body(snd3, rcv3, nf, posx, posy, posz,
                    msg_out, vx_out, vy_out, vz_out,
                    sidx, ridx, rows, dxb, dyb, dzb, px, py, pz, sem):
    w = lax.axis_index("s") * _NC + lax.axis_index("c")
    pltpu.sync_copy(posx, px)
    pltpu.sync_copy(posy, py)
    pltpu.sync_copy(posz, pz)

    def batch(j, carry):
        b = j * _NW + w

        @pl.when(b < _NBATCH)
        def _():
            pltpu.sync_copy(snd3.at[b], sidx)
            pltpu.sync_copy(rcv3.at[b], ridx)
            cps = [
                pltpu.async_copy(nf.at[sidx.at[k]],
                                 rows.at[pl.ds(k * _GCH, _GCH), :], sem)
                for k in range(_BCH)
            ]
            for k in range(_BCH):
                for g in range(_GCH // 16):
                    sl = pl.ds(g * 16, 16)
                    iv_s = sidx[k, sl]
                    iv_r = ridx[k, sl]
                    dxb[k, 0, sl] = (plsc.load_gather(px, [iv_r])
                                     - plsc.load_gather(px, [iv_s]))
                    dyb[k, 0, sl] = (plsc.load_gather(py, [iv_r])
                                     - plsc.load_gather(py, [iv_s]))
                    dzb[k, 0, sl] = (plsc.load_gather(pz, [iv_r])
                                     - plsc.load_gather(pz, [iv_s]))
            for cp in cps:
                cp.wait()
            pltpu.sync_copy(rows, msg_out.at[pl.ds(b * _BED, _BED), :])
            pltpu.sync_copy(dxb, vx_out.at[pl.ds(b * _BCH, _BCH)])
            pltpu.sync_copy(dyb, vy_out.at[pl.ds(b * _BCH, _BCH)])
            pltpu.sync_copy(dzb, vz_out.at[pl.ds(b * _BCH, _BCH)])

        return carry

    lax.fori_loop(0, _JMAX, batch, 0)


def _make_sc_gather():
    return pl.kernel(
        _sc_gather_body,
        out_type=(
            jax.ShapeDtypeStruct((_EP, _D), jnp.float32),
            jax.ShapeDtypeStruct((_NCHUNK, 1, _GCH), jnp.float32),
            jax.ShapeDtypeStruct((_NCHUNK, 1, _GCH), jnp.float32),
            jax.ShapeDtypeStruct((_NCHUNK, 1, _GCH), jnp.float32),
        ),
        mesh=_mesh(),
        scratch_types=[
            pltpu.VMEM((_BCH, _GCH), jnp.int32),
            pltpu.VMEM((_BCH, _GCH), jnp.int32),
            pltpu.VMEM((_BED, _D), jnp.float32),
            pltpu.VMEM((_BCH, 1, _GCH), jnp.float32),
            pltpu.VMEM((_BCH, 1, _GCH), jnp.float32),
            pltpu.VMEM((_BCH, 1, _GCH), jnp.float32),
            pltpu.VMEM((_N,), jnp.float32),
            pltpu.VMEM((_N,), jnp.float32),
            pltpu.VMEM((_N,), jnp.float32),
            pltpu.SemaphoreType.DMA,
        ],
        compiler_params=pltpu.CompilerParams(needs_layout_passes=False),
    )


def _sc_scatter_body(contrib, rcv1, zeros, out,
                     ridx0, ridx1, rows0, rows1, accum, sem):
    c = lax.axis_index("c")
    s = lax.axis_index("s")
    w = s * _NC + c
    ridxs = [ridx0, ridx1]
    rowss = [rows0, rows1]
    pltpu.sync_copy(zeros.at[pl.ds(s * _NPT, _NPT), :],
                    accum.at[pl.ds(s * _NPT, _NPT), :])
    plsc.subcore_barrier()

    def batch(j, carry):
        b = j * _NW + w

        @pl.when(b < _NBATCH)
        def _():
            off0 = b * _BED
            cps = [pltpu.async_copy(rcv1.at[pl.ds(off0, _GCH)], ridxs[0], sem),
                   pltpu.async_copy(contrib.at[pl.ds(off0, _GCH), :],
                                    rowss[0], sem)]
            for k in range(_BCH):
                nxt = (k + 1) % 2
                cur = k % 2
                cps_next = []
                if k + 1 < _BCH:
                    off = (b * _BCH + k + 1) * _GCH
                    cps_next = [
                        pltpu.async_copy(rcv1.at[pl.ds(off, _GCH)],
                                         ridxs[nxt], sem),
                        pltpu.async_copy(contrib.at[pl.ds(off, _GCH), :],
                                         rowss[nxt], sem),
                    ]
                for cp in cps:
                    cp.wait()
                cps = cps_next
                pltpu.sync_copy(rowss[cur], accum.at[ridxs[cur]], add=True)
        return carry

    lax.fori_loop(0, _JMAX, batch, 0)
    plsc.subcore_barrier()
    pltpu.sync_copy(accum.at[pl.ds(s * _NPT, _NPT), :],
                    out.at[c, pl.ds(s * _NPT, _NPT), :])


def _make_sc_scatter():
    return pl.kernel(
        _sc_scatter_body,
        out_type=jax.ShapeDtypeStruct((_NC, _NPAD, _D), jnp.float32),
        mesh=_mesh(),
        scratch_types=(
            [pltpu.VMEM((_GCH,), jnp.int32)] * 2
            + [pltpu.VMEM((_GCH, _D), jnp.float32)] * 2
            + [pltpu.VMEM_SHARED((_NPAD, _D), jnp.float32),
               pltpu.SemaphoreType.DMA]
        ),
    )


def _tc_up_body(x_ref, w_ref, o_ref):
    o_ref[...] = jnp.dot(x_ref[...], w_ref[...],
                         preferred_element_type=jnp.float32) * (1.0 / math.sqrt(_D))


def _tc_edge_body(cut_ref, msg_ref, vx_ref, vy_ref, vz_ref,
                  w1t_ref, w2t_ref, wdt_ref, o_ref):
    cutoff = cut_ref[0]
    nrow = _BE // _GCH
    vxl = vx_ref[:, 0, :]
    vyl = vy_ref[:, 0, :]
    vzl = vz_ref[:, 0, :]
    rl = jnp.sqrt(vxl * vxl + vyl * vyl + vzl * vzl)
    ul = math.sqrt(3.0) / jnp.where(rl > 1e-9, rl, 1.0)
    uxl = vxl * ul
    uyl = vyl * ul
    uzl = vzl * ul
    rrow = jnp.concatenate([rl[j:j + 1, :] for j in range(nrow)], axis=1)
    ux = jnp.concatenate([uxl[j:j + 1, :] for j in range(nrow)], axis=1)
    uy = jnp.concatenate([uyl[j:j + 1, :] for j in range(nrow)], axis=1)
    uz = jnp.concatenate([uzl[j:j + 1, :] for j in range(nrow)], axis=1)

    ks = lax.broadcasted_iota(jnp.int32, (_NRB, 1), 0).astype(jnp.float32) + 1.0
    safe = jnp.where(rrow > 1e-9, rrow, 1e-9)
    sq2c = jnp.sqrt(2.0 / cutoff)
    b = sq2c * jnp.sin(ks * (jnp.pi / cutoff) * rrow) / safe
    lim = (sq2c * (jnp.pi / cutoff)) * ks
    radial = jnp.where(rrow > 1e-9, b, jnp.broadcast_to(lim, b.shape))

    cste = 1.2 / math.exp(-1.0 / 2.0)
    y = 2.0 * (1.0 - rrow / cutoff)
    ysafe = jnp.where(y > 0.0, y, 1.0)
    env = cste * jnp.where(y > 0.0, jnp.exp(-1.0 / ysafe), 0.0)
    radial = radial * env

    h = jax.nn.gelu(jnp.dot(w1t_ref[...], radial,
                            preferred_element_type=jnp.float32)
                    * (1.0 / math.sqrt(_NRB)))
    mix = jnp.dot(w2t_ref[...], h,
                  preferred_element_type=jnp.float32) * (1.0 / math.sqrt(_HID))

    mt = msg_ref[...].T
    at = jnp.concatenate([
        mt * mix[0 * _D:1 * _D, :],
        mt * mix[1 * _D:2 * _D, :] * ux,
        mt * mix[2 * _D:3 * _D, :] * uy,
        mt * mix[3 * _D:4 * _D, :] * uz,
    ], axis=0)
    ct = jnp.dot(wdt_ref[...], at,
                 preferred_element_type=jnp.float32) * (
        1.0 / (math.sqrt(_MSG) * math.sqrt(_AVG)))
    o_ref[...] = ct.T


def _tc_add_body(p_ref, o_ref):
    o_ref[...] = p_ref[0] + p_ref[1]


_BN = 2000
_BE = 1280


def _tc_up(x, w):
    return pl.pallas_call(
        _tc_up_body,
        grid=(_N // _BN,),
        in_specs=[
            pl.BlockSpec((_BN, _D), lambda i: (i, 0)),
            pl.BlockSpec((_D, _D), lambda i: (0, 0)),
        ],
        out_specs=pl.BlockSpec((_BN, _D), lambda i: (i, 0)),
        out_shape=jax.ShapeDtypeStruct((_N, _D), jnp.float32),
        compiler_params=pltpu.CompilerParams(
            dimension_semantics=("parallel",)),
    )(x, w)


def _tc_edge(cut, msg, vx, vy, vz, w1t, w2t, wdt):
    return pl.pallas_call(
        _tc_edge_body,
        grid=(_EP // _BE,),
        in_specs=[
            pl.BlockSpec(memory_space=pltpu.SMEM),
            pl.BlockSpec((_BE, _D), lambda i: (i, 0)),
            pl.BlockSpec((_BE // _GCH, 1, _GCH), lambda i: (i, 0, 0)),
            pl.BlockSpec((_BE // _GCH, 1, _GCH), lambda i: (i, 0, 0)),
            pl.BlockSpec((_BE // _GCH, 1, _GCH), lambda i: (i, 0, 0)),
            pl.BlockSpec((_HID, _NRB), lambda i: (0, 0)),
            pl.BlockSpec((_MSG, _HID), lambda i: (0, 0)),
            pl.BlockSpec((_D, _MSG), lambda i: (0, 0)),
        ],
        out_specs=pl.BlockSpec((_BE, _D), lambda i: (i, 0)),
        out_shape=jax.ShapeDtypeStruct((_EP, _D), jnp.float32),
        compiler_params=pltpu.CompilerParams(
            dimension_semantics=("parallel",)),
    )(cut, msg, vx, vy, vz, w1t, w2t, wdt)


def _tc_add(p):
    return pl.pallas_call(
        _tc_add_body,
        grid=(_N // _BN,),
        in_specs=[pl.BlockSpec((_NC, _BN, _D), lambda i: (0, i, 0))],
        out_specs=pl.BlockSpec((_BN, _D), lambda i: (i, 0)),
        out_shape=jax.ShapeDtypeStruct((_N, _D), jnp.float32),
        compiler_params=pltpu.CompilerParams(
            dimension_semantics=("parallel",)),
    )(p)


def kernel(positions, node_feats, senders, receivers, cutoff,
           W_up, W1, W2, W_down):
    f32 = jnp.float32
    posx = positions[:, 0].astype(f32)
    posy = positions[:, 1].astype(f32)
    posz = positions[:, 2].astype(f32)
    s32 = senders.astype(jnp.int32)
    r32 = receivers.astype(jnp.int32)
    npad = _EP - _E
    s3 = jnp.concatenate([s32, jnp.zeros((npad,), jnp.int32)]
                         ).reshape(_NBATCH, _BCH, _GCH)
    rg3 = jnp.concatenate([r32, jnp.zeros((npad,), jnp.int32)]
                          ).reshape(_NBATCH, _BCH, _GCH)
    rs1 = jnp.concatenate([r32, jnp.full((npad,), _NPAD - 1, jnp.int32)])

    t2 = W2[:, _D:].reshape(_HID, _D, 3)
    w2p = jnp.concatenate([W2[:, :_D], t2[:, :, 0], t2[:, :, 1], t2[:, :, 2]],
                          axis=1).astype(f32)
    td = W_down[_D:, :].reshape(_D, 3, _D)
    wdp = jnp.concatenate([W_down[:_D, :], td[:, 0, :], td[:, 1, :],
                           td[:, 2, :]], axis=0).astype(f32)

    nf = _tc_up(node_feats.astype(f32), W_up.astype(f32))
    msg, vx, vy, vz = _make_sc_gather()(s3, rg3, nf, posx, posy, posz)
    cut = jnp.asarray(cutoff, f32).reshape(1)
    contrib = _tc_edge(cut, msg, vx, vy, vz, W1.T.astype(f32), w2p.T, wdp.T)
    zeros = jnp.zeros((_NPAD, _D), f32)
    partials = _make_sc_scatter()(contrib, rs1, zeros)
    return _tc_add(partials)
